# baseline (speedup 1.0000x reference)
import functools

import jax
import jax.numpy as jnp
from jax.experimental import pallas as pl

BATCH = 16384
LATENT = 2
CAT = 51


@functools.cache
def _gumbel2d():
    eps = 1e-20
    u = jax.random.uniform(jax.random.key(1), (BATCH, LATENT, CAT),
                           dtype=jnp.float32)
    g = jnp.log(-jnp.log(u + eps) + eps)
    return g.reshape(BATCH, LATENT * CAT)


def _onehot_body(x_ref, g_ref, o_ref):
    z = x_ref[...] + g_ref[...]
    lanes = jax.lax.broadcasted_iota(jnp.int32, z.shape, 1)

    def seg_onehot(lo, hi):
        seg = z[:, lo:hi]
        m = jnp.max(seg, axis=1, keepdims=True)
        iota = jax.lax.broadcasted_iota(jnp.int32, seg.shape, 1)
        idx = jnp.min(jnp.where(seg == m, iota, CAT), axis=1, keepdims=True)
        return idx + lo

    idx_a = seg_onehot(0, CAT)
    idx_b = seg_onehot(CAT, LATENT * CAT)
    o_ref[...] = ((lanes == idx_a) | (lanes == idx_b)).astype(jnp.float32)


def kernel(logits, temperature):
    del temperature
    x = logits.reshape(BATCH, LATENT * CAT)
    g = _gumbel2d()
    blk = 2048
    spec = pl.BlockSpec((blk, LATENT * CAT), lambda i: (i, 0))
    return pl.pallas_call(
        _onehot_body,
        grid=(BATCH // blk,),
        in_specs=[spec, spec],
        out_specs=spec,
        out_shape=jax.ShapeDtypeStruct((BATCH, LATENT * CAT), jnp.float32),
    )(x, g)


# speedup vs baseline: 1.4747x; 1.4747x over previous
import functools

import jax
import jax.numpy as jnp
from jax.experimental import pallas as pl

BATCH = 16384
LATENT = 2
CAT = 51


@functools.cache
def _gumbel_t():
    eps = 1e-20
    u = jax.random.uniform(jax.random.key(1), (BATCH, LATENT, CAT),
                           dtype=jnp.float32)
    g = jnp.log(-jnp.log(u + eps) + eps)
    return jnp.transpose(g, (1, 2, 0))


def _onehot_body(x0_ref, x1_ref, g0_ref, g1_ref, o_ref):
    blk = x0_ref.shape[2]
    iota = jax.lax.broadcasted_iota(jnp.int32, (CAT, blk), 0)

    def onehot_t(x, g):
        z = x + g
        m = jnp.max(z, axis=0, keepdims=True)
        idx = jnp.min(jnp.where(z == m, iota, CAT), axis=0, keepdims=True)
        return (iota == idx).astype(jnp.float32)

    oh_t = jnp.concatenate([onehot_t(x0_ref[0], g0_ref[0]),
                            onehot_t(x1_ref[0], g1_ref[0])], axis=0)
    o_ref[...] = oh_t.T


def kernel(logits, temperature):
    del temperature
    xt = jnp.transpose(logits, (1, 2, 0))
    gt = _gumbel_t()
    blk = 2048
    spec0 = pl.BlockSpec((1, CAT, blk), lambda i: (0, 0, i))
    spec1 = pl.BlockSpec((1, CAT, blk), lambda i: (1, 0, i))
    return pl.pallas_call(
        _onehot_body,
        grid=(BATCH // blk,),
        in_specs=[spec0, spec1, spec0, spec1],
        out_specs=pl.BlockSpec((blk, LATENT * CAT), lambda i: (i, 0)),
        out_shape=jax.ShapeDtypeStruct((BATCH, LATENT * CAT), jnp.float32),
    )(xt, xt, gt, gt)
